# baseline (speedup 1.0000x reference)
import functools

import jax
import jax.numpy as jnp
from jax import lax
from jax.experimental import pallas as pl
from jax.experimental.pallas import tpu as pltpu
from jax.experimental.pallas import tpu_sc as plsc

N = 10000
NP = 10240
K = 16
BQ = 512
BR = 128
BIG = 1e30


def _knn_body(xq_ref, xt_ref, idx_ref):
    q = xq_ref[...]
    xt = xt_ref[...]
    dot = lax.dot_general(q, xt, (((1,), (0,)), ((), ())),
                          preferred_element_type=jnp.float32)
    sq_q = jnp.sum(q * q, axis=1, keepdims=True)
    sq_x = jnp.sum(xt * xt, axis=0, keepdims=True)
    d2 = sq_q - 2.0 * dot + sq_x
    col = lax.broadcasted_iota(jnp.int32, (BQ, NP), 1)
    d2 = jnp.where(col >= N, BIG, d2)
    lane = lax.broadcasted_iota(jnp.int32, (BQ, K), 1)
    acc = jnp.zeros((BQ, K), jnp.int32)
    for t in range(K + 1):
        j = jnp.argmin(d2, axis=1).astype(jnp.int32)
        if t > 0:
            acc = jnp.where(lane == (t - 1), j[:, None], acc)
        if t < K:
            d2 = jnp.where(col == j[:, None], BIG, d2)
    idx_ref[...] = acc


def _knn(xp, xt, r0, nr):
    d = xp.shape[1]
    b0 = r0 // BQ
    return pl.pallas_call(
        _knn_body,
        grid=(nr // BQ,),
        in_specs=[pl.BlockSpec((BQ, d), lambda i: (i + b0, 0)),
                  pl.BlockSpec((d, NP), lambda i: (0, 0))],
        out_specs=pl.BlockSpec((BQ, K), lambda i: (i, 0)),
        out_shape=jax.ShapeDtypeStruct((nr, K), jnp.int32),
    )(xp, xt)


def _sc_gather(idx_flat, src, nr):
    d = src.shape[1]
    info = plsc.get_sparse_core_info()
    nc, ns = info.num_cores, info.num_subcores
    nw = nc * ns
    rows_per_w = nr // nw
    B = 8
    nbatch = rows_per_w // B
    mesh = plsc.VectorSubcoreMesh(core_axis_name="c", subcore_axis_name="s")

    @functools.partial(
        pl.kernel, mesh=mesh,
        out_type=jax.ShapeDtypeStruct((nr * K, d), jnp.float32),
        scratch_types=[
            pltpu.VMEM((2, B * K), jnp.int32),
            pltpu.VMEM((2, B * K, d), jnp.float32),
            pltpu.SemaphoreType.DMA,
            pltpu.SemaphoreType.DMA,
        ],
    )
    def gath(idx_hbm, src_hbm, out_hbm, idx_v, gath_v, sem0, sem1):
        wid = lax.axis_index("s") * nc + lax.axis_index("c")
        base0 = wid * rows_per_w
        sems = (sem0, sem1)

        pltpu.sync_copy(idx_hbm.at[pl.ds(base0 * K, B * K)], idx_v.at[0])
        cps = [pltpu.async_copy(src_hbm.at[idx_v.at[0]], gath_v.at[0], sems[0]),
               None]
        for bi in range(nbatch):
            cur = bi % 2
            nxt = (bi + 1) % 2
            if bi + 1 < nbatch:
                ebase_n = (base0 + (bi + 1) * B) * K
                pltpu.sync_copy(idx_hbm.at[pl.ds(ebase_n, B * K)],
                                idx_v.at[nxt])
                cps[nxt] = pltpu.async_copy(src_hbm.at[idx_v.at[nxt]],
                                            gath_v.at[nxt], sems[nxt])
            cps[cur].wait()
            ebase = (base0 + bi * B) * K
            pltpu.sync_copy(gath_v.at[cur], out_hbm.at[pl.ds(ebase, B * K)])

    return gath(idx_flat, src)


def _edgeconv_body(xi_ref, xj_ref, w_ref, b_ref, o_ref):
    d = xi_ref.shape[1]
    dout = o_ref.shape[1]
    xi = xi_ref[...]
    xj = xj_ref[...]
    xib = jnp.reshape(jnp.broadcast_to(xi[:, None, :], (BR, K, d)), (BR * K, d))
    e = jnp.concatenate([xib, xj - xib], axis=1)
    h = lax.dot_general(e, w_ref[...], (((1,), (0,)), ((), ())),
                        preferred_element_type=jnp.float32) + b_ref[...]
    h = jnp.maximum(h, 0.0)
    o_ref[...] = jnp.mean(jnp.reshape(h, (BR, K, dout)), axis=1)


def _edgeconv(xp, xj_flat, w, b, r0, nr):
    d = xp.shape[1]
    dout = w.shape[1]
    b0 = r0 // BR
    return pl.pallas_call(
        _edgeconv_body,
        grid=(nr // BR,),
        in_specs=[pl.BlockSpec((BR, d), lambda i: (i + b0, 0)),
                  pl.BlockSpec((BR * K, d), lambda i: (i, 0)),
                  pl.BlockSpec((2 * d, dout), lambda i: (0, 0)),
                  pl.BlockSpec((1, dout), lambda i: (0, 0))],
        out_specs=pl.BlockSpec((BR, dout), lambda i: (i, 0)),
        out_shape=jax.ShapeDtypeStruct((nr, dout), jnp.float32),
    )(xp, xj_flat, w, b)


def _mlp_body(h_ref, w1_ref, b1_ref, w2_ref, b2_ref, w3_ref, b3_ref, o_ref):
    h = h_ref[...]
    z = jnp.maximum(
        lax.dot_general(h, w1_ref[...], (((1,), (0,)), ((), ())),
                        preferred_element_type=jnp.float32) + b1_ref[...], 0.0)
    z = jnp.maximum(
        lax.dot_general(z, w2_ref[...], (((1,), (0,)), ((), ())),
                        preferred_element_type=jnp.float32) + b2_ref[...], 0.0)
    o_ref[...] = lax.dot_general(z, w3_ref[...], (((1,), (0,)), ((), ())),
                                 preferred_element_type=jnp.float32) + b3_ref[...]


def _mlp(h, w1, b1, w2, b2, w3, b3):
    din = h.shape[1]
    d1 = w1.shape[1]
    d2 = w2.shape[1]
    d3 = w3.shape[1]
    return pl.pallas_call(
        _mlp_body,
        grid=(NP // BQ,),
        in_specs=[pl.BlockSpec((BQ, din), lambda i: (i, 0)),
                  pl.BlockSpec((din, d1), lambda i: (0, 0)),
                  pl.BlockSpec((1, d1), lambda i: (0, 0)),
                  pl.BlockSpec((d1, d2), lambda i: (0, 0)),
                  pl.BlockSpec((1, d2), lambda i: (0, 0)),
                  pl.BlockSpec((d2, d3), lambda i: (0, 0)),
                  pl.BlockSpec((1, d3), lambda i: (0, 0))],
        out_specs=pl.BlockSpec((BQ, d3), lambda i: (i, 0)),
        out_shape=jax.ShapeDtypeStruct((NP, d3), jnp.float32),
    )(h, w1, b1, w2, b2, w3, b3)


def kernel(x, params):
    dims_in = [3, 64, 128, 256]
    x0p = jnp.zeros((NP, 128), jnp.float32).at[:N, :3].set(x)
    feats_real = [x0p[:, :3]]
    xp = x0p
    for i in range(4):
        din = dims_in[i]
        w = params[f'W{i + 1}']
        dout = w.shape[1]
        dout_pad = max(dout, 128)
        dpad = xp.shape[1]
        wp = jnp.zeros((2 * dpad, dout_pad), jnp.float32)
        wp = wp.at[:din, :dout].set(w[:din])
        wp = wp.at[dpad:dpad + din, :dout].set(w[din:])
        b = params[f'b{i + 1}']
        if dout_pad != dout:
            b = jnp.zeros((dout_pad,), jnp.float32).at[:dout].set(b)
        idx = _knn(xp, xp.T, 0, NP)
        xj_flat = _sc_gather(idx.reshape(-1), xp, NP)
        xp = _edgeconv(xp, xj_flat, wp, b[None, :], 0, NP)
        feats_real.append(xp[:, :dout])
    h = jnp.concatenate(feats_real, axis=1)
    h = jnp.pad(h, ((0, 0), (0, 1024 - h.shape[1])))
    w1 = jnp.zeros((1024, 512), jnp.float32).at[:963].set(params['mW1'])
    w3 = jnp.zeros((256, 128), jnp.float32).at[:, :3].set(params['mW3'])
    b3 = jnp.zeros((1, 128), jnp.float32).at[0, :3].set(params['mb3'])
    out = _mlp(h, w1, params['mb1'][None, :],
               params['mW2'], params['mb2'][None, :], w3, b3)
    return out[:N, :3]


# speedup vs baseline: 1.0081x; 1.0081x over previous
import functools

import jax
import jax.numpy as jnp
from jax import lax
from jax.experimental import pallas as pl
from jax.experimental.pallas import tpu as pltpu
from jax.experimental.pallas import tpu_sc as plsc

N = 10000
NP = 10240
K = 16
BQ = 512
BR = 256
BIG = 1e30


def _knn_body(xq_ref, xt_ref, idx_ref):
    q = xq_ref[...]
    xt = xt_ref[...]
    dot = lax.dot_general(q, xt, (((1,), (0,)), ((), ())),
                          preferred_element_type=jnp.float32)
    sq_q = jnp.sum(q * q, axis=1, keepdims=True)
    sq_x = jnp.sum(xt * xt, axis=0, keepdims=True)
    d2 = sq_q - 2.0 * dot + sq_x
    col = lax.broadcasted_iota(jnp.int32, (BQ, NP), 1)
    d2 = jnp.where(col >= N, BIG, d2)
    lane = lax.broadcasted_iota(jnp.int32, (BQ, K), 1)
    acc = jnp.zeros((BQ, K), jnp.int32)
    for t in range(K + 1):
        j = jnp.argmin(d2, axis=1).astype(jnp.int32)
        if t > 0:
            acc = jnp.where(lane == (t - 1), j[:, None], acc)
        if t < K:
            d2 = jnp.where(col == j[:, None], BIG, d2)
    idx_ref[...] = acc


def _knn(xp, xt, r0, nr):
    d = xp.shape[1]
    b0 = r0 // BQ
    return pl.pallas_call(
        _knn_body,
        grid=(nr // BQ,),
        in_specs=[pl.BlockSpec((BQ, d), lambda i: (i + b0, 0)),
                  pl.BlockSpec((d, NP), lambda i: (0, 0))],
        out_specs=pl.BlockSpec((BQ, K), lambda i: (i, 0)),
        out_shape=jax.ShapeDtypeStruct((nr, K), jnp.int32),
    )(xp, xt)


def _sc_gather(idx_flat, src, nr):
    d = src.shape[1]
    info = plsc.get_sparse_core_info()
    nc, ns = info.num_cores, info.num_subcores
    nw = nc * ns
    rows_per_w = nr // nw
    B = 8
    nbatch = rows_per_w // B
    mesh = plsc.VectorSubcoreMesh(core_axis_name="c", subcore_axis_name="s")

    @functools.partial(
        pl.kernel, mesh=mesh,
        out_type=jax.ShapeDtypeStruct((nr * K, d), jnp.float32),
        scratch_types=[
            pltpu.VMEM((rows_per_w * K,), jnp.int32),
            pltpu.VMEM((2, B * K, d), jnp.float32),
            pltpu.SemaphoreType.DMA,
            pltpu.SemaphoreType.DMA,
        ],
    )
    def gath(idx_hbm, src_hbm, out_hbm, idx_v, gath_v, sem0, sem1):
        wid = lax.axis_index("s") * nc + lax.axis_index("c")
        base0 = wid * rows_per_w
        sems = (sem0, sem1)

        pltpu.sync_copy(idx_hbm.at[pl.ds(base0 * K, rows_per_w * K)], idx_v)
        cps = [pltpu.async_copy(src_hbm.at[idx_v.at[pl.ds(0, B * K)]],
                                gath_v.at[0], sems[0]), None]
        for bi in range(nbatch):
            cur = bi % 2
            nxt = (bi + 1) % 2
            if bi + 1 < nbatch:
                cps[nxt] = pltpu.async_copy(
                    src_hbm.at[idx_v.at[pl.ds((bi + 1) * B * K, B * K)]],
                    gath_v.at[nxt], sems[nxt])
            cps[cur].wait()
            ebase = (base0 + bi * B) * K
            pltpu.sync_copy(gath_v.at[cur], out_hbm.at[pl.ds(ebase, B * K)])

    return gath(idx_flat, src)


def _edgeconv_body(xi_ref, xj_ref, w_ref, b_ref, o_ref):
    d = xi_ref.shape[1]
    dout = o_ref.shape[1]
    xi = xi_ref[...]
    xj = xj_ref[...]
    xib = jnp.reshape(jnp.broadcast_to(xi[:, None, :], (BR, K, d)), (BR * K, d))
    e = jnp.concatenate([xib, xj - xib], axis=1)
    h = lax.dot_general(e, w_ref[...], (((1,), (0,)), ((), ())),
                        preferred_element_type=jnp.float32) + b_ref[...]
    h = jnp.maximum(h, 0.0)
    o_ref[...] = jnp.mean(jnp.reshape(h, (BR, K, dout)), axis=1)


def _edgeconv(xp, xj_flat, w, b, r0, nr):
    d = xp.shape[1]
    dout = w.shape[1]
    b0 = r0 // BR
    return pl.pallas_call(
        _edgeconv_body,
        grid=(nr // BR,),
        in_specs=[pl.BlockSpec((BR, d), lambda i: (i + b0, 0)),
                  pl.BlockSpec((BR * K, d), lambda i: (i, 0)),
                  pl.BlockSpec((2 * d, dout), lambda i: (0, 0)),
                  pl.BlockSpec((1, dout), lambda i: (0, 0))],
        out_specs=pl.BlockSpec((BR, dout), lambda i: (i, 0)),
        out_shape=jax.ShapeDtypeStruct((nr, dout), jnp.float32),
    )(xp, xj_flat, w, b)


def _mlp_body(h_ref, w1_ref, b1_ref, w2_ref, b2_ref, w3_ref, b3_ref, o_ref):
    h = h_ref[...]
    z = jnp.maximum(
        lax.dot_general(h, w1_ref[...], (((1,), (0,)), ((), ())),
                        preferred_element_type=jnp.float32) + b1_ref[...], 0.0)
    z = jnp.maximum(
        lax.dot_general(z, w2_ref[...], (((1,), (0,)), ((), ())),
                        preferred_element_type=jnp.float32) + b2_ref[...], 0.0)
    o_ref[...] = lax.dot_general(z, w3_ref[...], (((1,), (0,)), ((), ())),
                                 preferred_element_type=jnp.float32) + b3_ref[...]


def _mlp(h, w1, b1, w2, b2, w3, b3):
    din = h.shape[1]
    d1 = w1.shape[1]
    d2 = w2.shape[1]
    d3 = w3.shape[1]
    return pl.pallas_call(
        _mlp_body,
        grid=(NP // BQ,),
        in_specs=[pl.BlockSpec((BQ, din), lambda i: (i, 0)),
                  pl.BlockSpec((din, d1), lambda i: (0, 0)),
                  pl.BlockSpec((1, d1), lambda i: (0, 0)),
                  pl.BlockSpec((d1, d2), lambda i: (0, 0)),
                  pl.BlockSpec((1, d2), lambda i: (0, 0)),
                  pl.BlockSpec((d2, d3), lambda i: (0, 0)),
                  pl.BlockSpec((1, d3), lambda i: (0, 0))],
        out_specs=pl.BlockSpec((BQ, d3), lambda i: (i, 0)),
        out_shape=jax.ShapeDtypeStruct((NP, d3), jnp.float32),
    )(h, w1, b1, w2, b2, w3, b3)


def kernel(x, params):
    dims_in = [3, 64, 128, 256]
    x0p = jnp.zeros((NP, 128), jnp.float32).at[:N, :3].set(x)
    feats_real = [x0p[:, :3]]
    xp = x0p
    for i in range(4):
        din = dims_in[i]
        w = params[f'W{i + 1}']
        dout = w.shape[1]
        dout_pad = max(dout, 128)
        dpad = xp.shape[1]
        wp = jnp.zeros((2 * dpad, dout_pad), jnp.float32)
        wp = wp.at[:din, :dout].set(w[:din])
        wp = wp.at[dpad:dpad + din, :dout].set(w[din:])
        b = params[f'b{i + 1}']
        if dout_pad != dout:
            b = jnp.zeros((dout_pad,), jnp.float32).at[:dout].set(b)
        idx = _knn(xp, xp.T, 0, NP)
        xj_flat = _sc_gather(idx.reshape(-1), xp, NP)
        xp = _edgeconv(xp, xj_flat, wp, b[None, :], 0, NP)
        feats_real.append(xp[:, :dout])
    h = jnp.concatenate(feats_real, axis=1)
    h = jnp.pad(h, ((0, 0), (0, 1024 - h.shape[1])))
    w1 = jnp.zeros((1024, 512), jnp.float32).at[:963].set(params['mW1'])
    w3 = jnp.zeros((256, 128), jnp.float32).at[:, :3].set(params['mW3'])
    b3 = jnp.zeros((1, 128), jnp.float32).at[0, :3].set(params['mb3'])
    out = _mlp(h, w1, params['mb1'][None, :],
               params['mW2'], params['mb2'][None, :], w3, b3)
    return out[:N, :3]
